# baseline (speedup 1.0000x reference)
import functools

import jax
import jax.numpy as jnp
from jax import lax
from jax.experimental import pallas as pl
from jax.experimental.pallas import tpu as pltpu
from jax.experimental.pallas import tpu_sc as plsc

D = 128
CB = 128
NW = 32
N_SUB = 16


def _sc_aggregate(h, zeros_pad, pk_t, ch0, ch1, npad):
    chmax = max(ch0, ch1)
    rows_per_tile = npad // N_SUB
    mesh = plsc.VectorSubcoreMesh(core_axis_name="c", subcore_axis_name="s")

    @functools.partial(
        pl.kernel,
        out_type=jax.ShapeDtypeStruct((2, npad, D), jnp.float32),
        mesh=mesh,
        scratch_types=[
            pltpu.VMEM((chmax, CB), jnp.int32),
            pltpu.VMEM((2, CB), jnp.int32),
            pltpu.VMEM((2, CB), jnp.int32),
            pltpu.VMEM((2, CB, D), jnp.float32),
            pltpu.VMEM_SHARED((npad, D), jnp.float32),
            pltpu.SemaphoreType.DMA,
            pltpu.SemaphoreType.DMA,
        ],
    )
    def agg(h_hbm, z_hbm, pk_hbm, out_hbm,
            pk_v, sidx_v, didx_v, rows_v, acc, gsem, ssem):
        cid = lax.axis_index("c")
        sid = lax.axis_index("s")
        wid = cid * N_SUB + sid
        r0 = sid * rows_per_tile
        pltpu.sync_copy(z_hbm.at[pl.ds(r0, rows_per_tile)],
                        acc.at[pl.ds(r0, rows_per_tile)])
        pltpu.sync_copy(pk_hbm.at[wid], pk_v)
        plsc.subcore_barrier()

        def unpack(j, p):
            for k in range(CB // 16):
                v = pk_v[j, pl.ds(k * 16, 16)]
                sidx_v[p, pl.ds(k * 16, 16)] = lax.shift_right_logical(v, 16)
                didx_v[p, pl.ds(k * 16, 16)] = lax.bitwise_and(v, 0xFFFF)

        def gather(b):
            return pltpu.make_async_copy(
                h_hbm.at[sidx_v.at[b]], rows_v.at[b], gsem)

        def scatter(b):
            return pltpu.make_async_copy(
                rows_v.at[b], acc.at[didx_v.at[b]], ssem)

        my_ch = jnp.where(cid == 0, ch0, ch1)
        unpack(0, 0)
        gather(0).start()

        def pair(i, carry):
            j = 2 * i
            gather(0).wait()
            scatter(0).start(add=True)

            @pl.when(j >= 1)
            def _():
                scatter(1).wait()

            unpack(j + 1, 1)
            gather(1).start()
            gather(1).wait()
            scatter(1).start(add=True)
            scatter(0).wait()

            @pl.when(j + 2 < my_ch)
            def _():
                unpack(j + 2, 0)
                gather(0).start()

            return carry

        lax.fori_loop(0, my_ch // 2, pair, 0)
        scatter(1).wait()
        plsc.subcore_barrier()
        pltpu.sync_copy(acc.at[pl.ds(r0, rows_per_tile)],
                        out_hbm.at[cid, pl.ds(r0, rows_per_tile)])

    return agg(h, zeros_pad, pk_t)


def _mlp_call(partials, h, Wa, ba, Wb, bb, final_relu):
    n = h.shape[0]
    br = 1000
    grid = (n // br,)

    def body(p_ref, h_ref, wa_ref, ba_ref, wb_ref, bb_ref, o_ref):
        a = h_ref[...] + p_ref[0] + p_ref[1]
        t = jnp.dot(a, wa_ref[...], preferred_element_type=jnp.float32)
        t = jnp.maximum(t + ba_ref[...], 0.0)
        t = jnp.dot(t, wb_ref[...], preferred_element_type=jnp.float32)
        t = t + bb_ref[...]
        if final_relu:
            t = jnp.maximum(t, 0.0)
        o_ref[...] = t

    return pl.pallas_call(
        body,
        grid=grid,
        in_specs=[
            pl.BlockSpec((2, br, D), lambda i: (0, i, 0)),
            pl.BlockSpec((br, D), lambda i: (i, 0)),
            pl.BlockSpec((D, D), lambda i: (0, 0)),
            pl.BlockSpec((1, D), lambda i: (0, 0)),
            pl.BlockSpec((D, D), lambda i: (0, 0)),
            pl.BlockSpec((1, D), lambda i: (0, 0)),
        ],
        out_specs=pl.BlockSpec((br, D), lambda i: (i, 0)),
        out_shape=jax.ShapeDtypeStruct((n, D), jnp.float32),
    )(partials, h, Wa, ba.reshape(1, D), Wb, bb.reshape(1, D))


def kernel(x, edge_index, W1a, b1a, W1b, b1b, W2a, b2a, W2b, b2b):
    n = x.shape[0]
    npad = ((n + 127) // 128) * 128 + 128 if n % 128 == 0 else -(-n // 128) * 128
    src = edge_index[0].astype(jnp.int32)
    dst = edge_index[1].astype(jnp.int32)
    e = src.shape[0]
    f_fast = 0.63
    chunks = -(-e // CB)
    ch0 = max(2, min(int(round(f_fast * chunks / N_SUB)), chunks // N_SUB))
    ch0 += ch0 % 2
    ch1 = max(2, -(-(chunks - N_SUB * ch0) // N_SUB))
    ch1 += ch1 % 2
    chmax = max(ch0, ch1)
    cap0 = N_SUB * ch0 * CB
    cap1 = N_SUB * ch1 * CB
    pk = (src << 16) | dst
    pk_f = jnp.concatenate([pk, jnp.full((cap0 + cap1 - e,), n, jnp.int32)])
    a0 = pk_f[:cap0].reshape(N_SUB, ch0, CB)
    a1 = pk_f[cap0:].reshape(N_SUB, ch1, CB)
    a0 = jnp.pad(a0, ((0, 0), (0, chmax - ch0), (0, 0)), constant_values=n)
    a1 = jnp.pad(a1, ((0, 0), (0, chmax - ch1), (0, 0)), constant_values=n)
    pk_p = jnp.concatenate([a0, a1])
    zeros_pad = jnp.zeros((npad, D), jnp.float32)

    p1 = _sc_aggregate(x, zeros_pad, pk_p, ch0, ch1, npad)
    h1 = _mlp_call(p1, x, W1a, b1a, W1b, b1b, final_relu=True)
    p2 = _sc_aggregate(h1, zeros_pad, pk_p, ch0, ch1, npad)
    out = _mlp_call(p2, h1, W2a, b2a, W2b, b2b, final_relu=False)
    return out


# speedup vs baseline: 1.1683x; 1.1683x over previous
import functools

import jax
import jax.numpy as jnp
from jax import lax
from jax.experimental import pallas as pl
from jax.experimental.pallas import tpu as pltpu
from jax.experimental.pallas import tpu_sc as plsc

D = 128
CB = 128
NW = 32
N_SUB = 16


def _sc_aggregate(h, zeros_pad, src_t, dst_t, ch0, ch1, npad):
    chmax = max(ch0, ch1)
    rows_per_tile = npad // N_SUB
    mesh = plsc.VectorSubcoreMesh(core_axis_name="c", subcore_axis_name="s")

    @functools.partial(
        pl.kernel,
        out_type=jax.ShapeDtypeStruct((2, npad, D), jnp.float32),
        mesh=mesh,
        scratch_types=[
            pltpu.VMEM((chmax, CB), jnp.int32),
            pltpu.VMEM((chmax, CB), jnp.int32),
            pltpu.VMEM((CB, D), jnp.float32),
            pltpu.VMEM_SHARED((npad, D), jnp.float32),
            pltpu.SemaphoreType.DMA,
        ],
    )
    def agg(h_hbm, z_hbm, src_hbm, dst_hbm, out_hbm,
            src_v, dst_v, rows_v, acc, sem):
        cid = lax.axis_index("c")
        sid = lax.axis_index("s")
        wid = cid * N_SUB + sid
        r0 = sid * rows_per_tile
        pltpu.sync_copy(z_hbm, acc.at[pl.ds(r0, rows_per_tile)])
        pltpu.sync_copy(src_hbm.at[wid], src_v)
        pltpu.sync_copy(dst_hbm.at[wid], dst_v)
        plsc.subcore_barrier()

        def body(j, carry):
            pltpu.async_copy(h_hbm.at[src_v.at[j]], rows_v, sem).wait()
            pltpu.sync_copy(rows_v, acc.at[dst_v.at[j]], add=True)
            return carry

        my_ch = jnp.where(cid == 0, ch0, ch1)
        lax.fori_loop(0, my_ch, body, 0)
        plsc.subcore_barrier()
        pltpu.sync_copy(acc.at[pl.ds(r0, rows_per_tile)],
                        out_hbm.at[cid, pl.ds(r0, rows_per_tile)])

    return agg(h, zeros_pad, src_t, dst_t)


def _mlp_call(partials, h, Wa, ba, Wb, bb, final_relu):
    n = h.shape[0]
    br = 1000
    grid = (n // br,)

    def body(p_ref, h_ref, wa_ref, ba_ref, wb_ref, bb_ref, o_ref):
        a = h_ref[...] + p_ref[0] + p_ref[1]
        t = jnp.dot(a, wa_ref[...], preferred_element_type=jnp.float32)
        t = jnp.maximum(t + ba_ref[...], 0.0)
        t = jnp.dot(t, wb_ref[...], preferred_element_type=jnp.float32)
        t = t + bb_ref[...]
        if final_relu:
            t = jnp.maximum(t, 0.0)
        o_ref[...] = t

    return pl.pallas_call(
        body,
        grid=grid,
        in_specs=[
            pl.BlockSpec((2, br, D), lambda i: (0, i, 0)),
            pl.BlockSpec((br, D), lambda i: (i, 0)),
            pl.BlockSpec((D, D), lambda i: (0, 0)),
            pl.BlockSpec((1, D), lambda i: (0, 0)),
            pl.BlockSpec((D, D), lambda i: (0, 0)),
            pl.BlockSpec((1, D), lambda i: (0, 0)),
        ],
        out_specs=pl.BlockSpec((br, D), lambda i: (i, 0)),
        out_shape=jax.ShapeDtypeStruct((n, D), jnp.float32),
    )(partials, h, Wa, ba.reshape(1, D), Wb, bb.reshape(1, D))


def kernel(x, edge_index, W1a, b1a, W1b, b1b, W2a, b2a, W2b, b2b):
    n = x.shape[0]
    npad = ((n + 127) // 128) * 128 + 128 if n % 128 == 0 else -(-n // 128) * 128
    src = edge_index[0].astype(jnp.int32)
    dst = edge_index[1].astype(jnp.int32)
    e = src.shape[0]
    f_fast = 0.63
    chunks = -(-e // CB)
    ch0 = max(1, min(int(round(f_fast * chunks / N_SUB)), chunks // N_SUB))
    ch1 = max(1, -(-(chunks - N_SUB * ch0) // N_SUB))
    chmax = max(ch0, ch1)
    cap0 = N_SUB * ch0 * CB
    cap1 = N_SUB * ch1 * CB
    src_f = jnp.concatenate([src, jnp.zeros((cap0 + cap1 - e,), jnp.int32)])
    dst_f = jnp.concatenate([dst, jnp.full((cap0 + cap1 - e,), n, jnp.int32)])

    def per_tile(flat, fill):
        a0 = flat[:cap0].reshape(N_SUB, ch0, CB)
        a1 = flat[cap0:].reshape(N_SUB, ch1, CB)
        a0 = jnp.pad(a0, ((0, 0), (0, chmax - ch0), (0, 0)),
                     constant_values=fill)
        a1 = jnp.pad(a1, ((0, 0), (0, chmax - ch1), (0, 0)),
                     constant_values=fill)
        return jnp.concatenate([a0, a1])

    src_p = per_tile(src_f, 0)
    dst_p = per_tile(dst_f, n)
    zeros_pad = jnp.zeros((npad // N_SUB, D), jnp.float32)

    p1 = _sc_aggregate(x, zeros_pad, src_p, dst_p, ch0, ch1, npad)
    h1 = _mlp_call(p1, x, W1a, b1a, W1b, b1b, final_relu=True)
    p2 = _sc_aggregate(h1, zeros_pad, src_p, dst_p, ch0, ch1, npad)
    out = _mlp_call(p2, h1, W2a, b2a, W2b, b2b, final_relu=False)
    return out


# speedup vs baseline: 1.1741x; 1.0050x over previous
import functools

import jax
import jax.numpy as jnp
from jax import lax
from jax.experimental import pallas as pl
from jax.experimental.pallas import tpu as pltpu
from jax.experimental.pallas import tpu_sc as plsc

D = 128
CB = 128
NW = 32
N_SUB = 16


def _sc_aggregate(h, zeros_pad, src_t, dst_t, ch0, ch1, npad):
    chmax = max(ch0, ch1)
    rows_per_tile = npad // N_SUB
    mesh = plsc.VectorSubcoreMesh(core_axis_name="c", subcore_axis_name="s")

    @functools.partial(
        pl.kernel,
        out_type=jax.ShapeDtypeStruct((2, npad, D), jnp.float32),
        mesh=mesh,
        scratch_types=[
            pltpu.VMEM((chmax, CB), jnp.int32),
            pltpu.VMEM((chmax, CB), jnp.int32),
            pltpu.VMEM((CB, D), jnp.float32),
            pltpu.VMEM_SHARED((npad, D), jnp.float32),
            pltpu.SemaphoreType.DMA,
        ],
    )
    def agg(h_hbm, z_hbm, src_hbm, dst_hbm, out_hbm,
            src_v, dst_v, rows_v, acc, sem):
        cid = lax.axis_index("c")
        sid = lax.axis_index("s")
        wid = cid * N_SUB + sid
        r0 = sid * rows_per_tile
        pltpu.sync_copy(z_hbm.at[pl.ds(r0, rows_per_tile)],
                        acc.at[pl.ds(r0, rows_per_tile)])
        pltpu.sync_copy(src_hbm.at[wid], src_v)
        pltpu.sync_copy(dst_hbm.at[wid], dst_v)
        plsc.subcore_barrier()

        def body(j, carry):
            pltpu.async_copy(h_hbm.at[src_v.at[j]], rows_v, sem).wait()
            pltpu.sync_copy(rows_v, acc.at[dst_v.at[j]], add=True)
            return carry

        my_ch = jnp.where(cid == 0, ch0, ch1)
        lax.fori_loop(0, my_ch, body, 0)
        plsc.subcore_barrier()
        pltpu.sync_copy(acc.at[pl.ds(r0, rows_per_tile)],
                        out_hbm.at[cid, pl.ds(r0, rows_per_tile)])

    return agg(h, zeros_pad, src_t, dst_t)


def _mlp_call(partials, h, Wa, ba, Wb, bb, final_relu):
    n = h.shape[0]
    br = 1000
    grid = (n // br,)

    def body(p_ref, h_ref, wa_ref, ba_ref, wb_ref, bb_ref, o_ref):
        a = h_ref[...] + p_ref[0] + p_ref[1]
        t = jnp.dot(a, wa_ref[...], preferred_element_type=jnp.float32)
        t = jnp.maximum(t + ba_ref[...], 0.0)
        t = jnp.dot(t, wb_ref[...], preferred_element_type=jnp.float32)
        t = t + bb_ref[...]
        if final_relu:
            t = jnp.maximum(t, 0.0)
        o_ref[...] = t

    return pl.pallas_call(
        body,
        grid=grid,
        in_specs=[
            pl.BlockSpec((2, br, D), lambda i: (0, i, 0)),
            pl.BlockSpec((br, D), lambda i: (i, 0)),
            pl.BlockSpec((D, D), lambda i: (0, 0)),
            pl.BlockSpec((1, D), lambda i: (0, 0)),
            pl.BlockSpec((D, D), lambda i: (0, 0)),
            pl.BlockSpec((1, D), lambda i: (0, 0)),
        ],
        out_specs=pl.BlockSpec((br, D), lambda i: (i, 0)),
        out_shape=jax.ShapeDtypeStruct((n, D), jnp.float32),
    )(partials, h, Wa, ba.reshape(1, D), Wb, bb.reshape(1, D))


def kernel(x, edge_index, W1a, b1a, W1b, b1b, W2a, b2a, W2b, b2b):
    n = x.shape[0]
    npad = ((n + 127) // 128) * 128 + 128 if n % 128 == 0 else -(-n // 128) * 128
    src = edge_index[0].astype(jnp.int32)
    dst = edge_index[1].astype(jnp.int32)
    e = src.shape[0]
    f_fast = 0.63
    chunks = -(-e // CB)
    ch0 = max(1, min(int(round(f_fast * chunks / N_SUB)), chunks // N_SUB))
    ch1 = max(1, -(-(chunks - N_SUB * ch0) // N_SUB))
    chmax = max(ch0, ch1)
    cap0 = N_SUB * ch0 * CB
    cap1 = N_SUB * ch1 * CB
    src_f = jnp.concatenate([src, jnp.zeros((cap0 + cap1 - e,), jnp.int32)])
    dst_f = jnp.concatenate([dst, jnp.full((cap0 + cap1 - e,), n, jnp.int32)])

    def per_tile(flat, fill):
        a0 = flat[:cap0].reshape(N_SUB, ch0, CB)
        a1 = flat[cap0:].reshape(N_SUB, ch1, CB)
        a0 = jnp.pad(a0, ((0, 0), (0, chmax - ch0), (0, 0)),
                     constant_values=fill)
        a1 = jnp.pad(a1, ((0, 0), (0, chmax - ch1), (0, 0)),
                     constant_values=fill)
        return jnp.concatenate([a0, a1])

    src_p = per_tile(src_f, 0)
    dst_p = per_tile(dst_f, n)
    zeros_pad = jnp.zeros((npad, D), jnp.float32)

    p1 = _sc_aggregate(x, zeros_pad, src_p, dst_p, ch0, ch1, npad)
    h1 = _mlp_call(p1, x, W1a, b1a, W1b, b1b, final_relu=True)
    p2 = _sc_aggregate(h1, zeros_pad, src_p, dst_p, ch0, ch1, npad)
    out = _mlp_call(p2, h1, W2a, b2a, W2b, b2b, final_relu=False)
    return out
